# Initial kernel scaffold; baseline (speedup 1.0000x reference)
import jax
import jax.numpy as jnp
from jax.experimental import pallas as pl


def kernel(x, edge_index, W1l, W1r, b1, W2l, W2r, b2):
    raise NotImplementedError("write your pallas kernel here")


# speedup vs baseline: 4.0056x; 4.0056x over previous
import jax
import jax.numpy as jnp
from jax import lax
from jax.experimental import pallas as pl
from jax.experimental.pallas import tpu as pltpu
from jax.experimental.pallas import tpu_sc as plsc

N = 10000
D = 128
E = 320000
NC = 2
NS = 16
NW = NC * NS
CHUNK = 80
AR = 10240
RPT = AR // NS
EPT1 = E // NS
NCHUNK1 = EPT1 // CHUNK
EPW2 = E // NW
NCHUNK2 = EPW2 // CHUNK

_mesh = plsc.VectorSubcoreMesh(core_axis_name="c", subcore_axis_name="s")


def _fill(ref, nrows, val):
    v = jnp.full((16,), val, jnp.float32)

    def row(i, _):
        def col(j, _):
            ref[i, pl.ds(j * 16, 16)] = v
            return 0
        return lax.fori_loop(0, D // 16, col, 0)
    lax.fori_loop(0, nrows, row, 0)


def _zero_acc(acc, rows, s):
    _fill(rows, CHUNK, 0.0)
    for r in range(RPT // CHUNK):
        pltpu.sync_copy(rows, acc.at[pl.ds(s * RPT + r * CHUNK, CHUNK)])


def _agg1_body(feat, srci, dsti, out, acc, src_v, dst_v, rows, sem):
    c = lax.axis_index("c")
    s = lax.axis_index("s")

    _zero_acc(acc, rows, s)
    _fill(rows, CHUNK, 1.0)
    plsc.subcore_barrier()

    base = s * EPT1

    def chunk(ci, _):
        off = base + ci * CHUNK
        pltpu.sync_copy(dsti.at[pl.ds(off, CHUNK)], dst_v)

        @pl.when(c == 0)
        def _():
            pltpu.sync_copy(srci.at[pl.ds(off, CHUNK)], src_v)
            pltpu.async_copy(feat.at[src_v], rows, sem).wait()

        pltpu.sync_copy(rows, acc.at[dst_v], add=True)
        return 0
    lax.fori_loop(0, NCHUNK1, chunk, 0)

    plsc.subcore_barrier()
    pltpu.sync_copy(acc.at[pl.ds(s * RPT, RPT)],
                    out.at[c, pl.ds(s * RPT, RPT)])


_agg1 = pl.kernel(
    _agg1_body,
    out_type=jax.ShapeDtypeStruct((NC, AR, D), jnp.float32),
    mesh=_mesh,
    scratch_types=(
        pltpu.VMEM_SHARED((AR, D), jnp.float32),
        pltpu.VMEM((CHUNK,), jnp.int32),
        pltpu.VMEM((CHUNK,), jnp.int32),
        pltpu.VMEM((CHUNK, D), jnp.float32),
        pltpu.SemaphoreType.DMA,
    ),
)


def _agg2_body(feat, srci, dsti, out, acc, src_v, dst_v, rows, sem):
    c = lax.axis_index("c")
    s = lax.axis_index("s")
    wid = c * NS + s

    _zero_acc(acc, rows, s)
    plsc.subcore_barrier()

    base = wid * EPW2

    def chunk(ci, _):
        off = base + ci * CHUNK
        pltpu.sync_copy(srci.at[pl.ds(off, CHUNK)], src_v)
        pltpu.sync_copy(dsti.at[pl.ds(off, CHUNK)], dst_v)
        pltpu.async_copy(feat.at[src_v], rows, sem).wait()
        pltpu.sync_copy(rows, acc.at[dst_v], add=True)
        return 0
    lax.fori_loop(0, NCHUNK2, chunk, 0)

    plsc.subcore_barrier()
    pltpu.sync_copy(acc.at[pl.ds(s * RPT, RPT)],
                    out.at[c, pl.ds(s * RPT, RPT)])


_agg2 = pl.kernel(
    _agg2_body,
    out_type=jax.ShapeDtypeStruct((NC, AR, D), jnp.float32),
    mesh=_mesh,
    scratch_types=(
        pltpu.VMEM_SHARED((AR, D), jnp.float32),
        pltpu.VMEM((CHUNK,), jnp.int32),
        pltpu.VMEM((CHUNK,), jnp.int32),
        pltpu.VMEM((CHUNK, D), jnp.float32),
        pltpu.SemaphoreType.DMA,
    ),
)

_BLK = 1000


def _make_dense(two_partials, relu):
    def body(p_ref, c_ref, x_ref, wl_ref, wr_ref, b_ref, o_ref):
        if two_partials:
            psum = p_ref[0] + p_ref[1]
        else:
            psum = p_ref[0]
        cnt = jnp.maximum(c_ref[0, :, 0:1], 1.0)
        mean = psum / cnt
        acc = lax.dot_general(mean, wl_ref[...], (((1,), (1,)), ((), ())),
                              preferred_element_type=jnp.float32)
        acc = acc + lax.dot_general(x_ref[...], wr_ref[...],
                                    (((1,), (1,)), ((), ())),
                                    preferred_element_type=jnp.float32)
        acc = acc + b_ref[...]
        if relu:
            acc = jnp.maximum(acc, 0.0)
        o_ref[...] = acc

    np = NC if two_partials else 1
    return pl.pallas_call(
        body,
        grid=(N // _BLK,),
        in_specs=[
            pl.BlockSpec((np, _BLK, D), lambda i: (0, i, 0)),
            pl.BlockSpec((1, _BLK, D), lambda i: (1, i, 0)),
            pl.BlockSpec((_BLK, D), lambda i: (i, 0)),
            pl.BlockSpec((D, D), lambda i: (0, 0)),
            pl.BlockSpec((D, D), lambda i: (0, 0)),
            pl.BlockSpec((1, D), lambda i: (0, 0)),
        ],
        out_specs=pl.BlockSpec((_BLK, D), lambda i: (i, 0)),
        out_shape=jax.ShapeDtypeStruct((N, D), jnp.float32),
    )


_dense1 = _make_dense(False, True)
_dense2 = _make_dense(True, False)


@jax.jit
def kernel(x, edge_index, W1l, W1r, b1, W2l, W2r, b2):
    src = edge_index[0].astype(jnp.int32)
    dst = edge_index[1].astype(jnp.int32)
    p1 = _agg1(x, src, dst)
    h = _dense1(p1[0:1], p1, x, W1l, W1r, b1.reshape(1, D))
    p2 = _agg2(h, src, dst)
    return _dense2(p2, p1, h, W2l, W2r, b2.reshape(1, D))


# speedup vs baseline: 6.6073x; 1.6495x over previous
import jax
import jax.numpy as jnp
from jax import lax
from jax.experimental import pallas as pl
from jax.experimental.pallas import tpu as pltpu
from jax.experimental.pallas import tpu_sc as plsc

N = 10000
D = 128
E = 320000
NC = 2
NS = 16
NW = NC * NS
CHUNK = 80
AR = 10240
RPT = AR // NS

_mesh = plsc.VectorSubcoreMesh(core_axis_name="c", subcore_axis_name="s")


def _fill(ref, nrows, val):
    v = jnp.full((16,), val, jnp.float32)

    def row(i, _):
        def col(j, _):
            ref[i, pl.ds(j * 16, 16)] = v
            return 0
        return lax.fori_loop(0, D // 16, col, 0)
    lax.fori_loop(0, nrows, row, 0)


def _make_agg(layer1):
    nchunk = (E // NS if layer1 else E // NW) // CHUNK

    def body(feat, srci, dsti, out, acc,
             src_v0, src_v1, dst_v0, dst_v1, rows0, rows1, sem0, sem1):
        c = lax.axis_index("c")
        s = lax.axis_index("s")
        src_v = (src_v0, src_v1)
        dst_v = (dst_v0, dst_v1)
        rows = (rows0, rows1)
        sem = (sem0, sem1)
        base = (s * (E // NS)) if layer1 else ((c * NS + s) * (E // NW))

        _fill(rows0, CHUNK, 0.0)
        for r in range(RPT // CHUNK):
            pltpu.sync_copy(rows0, acc.at[pl.ds(s * RPT + r * CHUNK, CHUNK)])
        if layer1:
            _fill(rows0, CHUNK, 1.0)
            _fill(rows1, CHUNK, 1.0)
        plsc.subcore_barrier()

        def start(ci, b):
            off = base + ci * CHUNK
            pltpu.sync_copy(dsti.at[pl.ds(off, CHUNK)], dst_v[b])

            def gath():
                pltpu.sync_copy(srci.at[pl.ds(off, CHUNK)], src_v[b])
                pltpu.async_copy(feat.at[src_v[b]], rows[b], sem[b])
            if layer1:
                pl.when(c == 0)(gath)
            else:
                gath()

        def finish(ci, b):
            def wait():
                pltpu.make_async_copy(feat.at[src_v[b]], rows[b],
                                      sem[b]).wait()
            if layer1:
                pl.when(c == 0)(wait)
            else:
                wait()
            pltpu.sync_copy(rows[b], acc.at[dst_v[b]], add=True)

        start(0, 0)

        @pl.loop(0, (nchunk // 2) * 2, step=2)
        def _(ci0):
            for b in range(2):
                ci = ci0 + b

                @pl.when(ci + 1 < nchunk)
                def _():
                    start(ci + 1, 1 - b)
                finish(ci, b)

        if nchunk % 2:
            finish(nchunk - 1, 0)

        plsc.subcore_barrier()
        pltpu.sync_copy(acc.at[pl.ds(s * RPT, RPT)],
                        out.at[c, pl.ds(s * RPT, RPT)])

    return pl.kernel(
        body,
        out_type=jax.ShapeDtypeStruct((NC, AR, D), jnp.float32),
        mesh=_mesh,
        scratch_types=(
            pltpu.VMEM_SHARED((AR, D), jnp.float32),
            pltpu.VMEM((CHUNK,), jnp.int32),
            pltpu.VMEM((CHUNK,), jnp.int32),
            pltpu.VMEM((CHUNK,), jnp.int32),
            pltpu.VMEM((CHUNK,), jnp.int32),
            pltpu.VMEM((CHUNK, D), jnp.float32),
            pltpu.VMEM((CHUNK, D), jnp.float32),
            pltpu.SemaphoreType.DMA,
            pltpu.SemaphoreType.DMA,
        ),
    )


_agg1 = _make_agg(True)
_agg2 = _make_agg(False)

_BLK = 1000


def _make_dense(two_partials, relu):
    def body(p_ref, c_ref, x_ref, wl_ref, wr_ref, b_ref, o_ref):
        if two_partials:
            psum = p_ref[0] + p_ref[1]
        else:
            psum = p_ref[0]
        cnt = jnp.maximum(c_ref[0, :, 0:1], 1.0)
        mean = psum / cnt
        acc = lax.dot_general(mean, wl_ref[...], (((1,), (1,)), ((), ())),
                              preferred_element_type=jnp.float32)
        acc = acc + lax.dot_general(x_ref[...], wr_ref[...],
                                    (((1,), (1,)), ((), ())),
                                    preferred_element_type=jnp.float32)
        acc = acc + b_ref[...]
        if relu:
            acc = jnp.maximum(acc, 0.0)
        o_ref[...] = acc

    np = NC if two_partials else 1
    return pl.pallas_call(
        body,
        grid=(N // _BLK,),
        in_specs=[
            pl.BlockSpec((np, _BLK, D), lambda i: (0, i, 0)),
            pl.BlockSpec((1, _BLK, D), lambda i: (1, i, 0)),
            pl.BlockSpec((_BLK, D), lambda i: (i, 0)),
            pl.BlockSpec((D, D), lambda i: (0, 0)),
            pl.BlockSpec((D, D), lambda i: (0, 0)),
            pl.BlockSpec((1, D), lambda i: (0, 0)),
        ],
        out_specs=pl.BlockSpec((_BLK, D), lambda i: (i, 0)),
        out_shape=jax.ShapeDtypeStruct((N, D), jnp.float32),
    )


_dense1 = _make_dense(False, True)
_dense2 = _make_dense(True, False)


@jax.jit
def kernel(x, edge_index, W1l, W1r, b1, W2l, W2r, b2):
    src = edge_index[0].astype(jnp.int32)
    dst = edge_index[1].astype(jnp.int32)
    p1 = _agg1(x, src, dst)
    h = _dense1(p1[0:1], p1, x, W1l, W1r, b1.reshape(1, D))
    p2 = _agg2(h, src, dst)
    return _dense2(p2, p1, h, W2l, W2r, b2.reshape(1, D))


# speedup vs baseline: 8.8057x; 1.3327x over previous
import jax
import jax.numpy as jnp
from jax import lax
from jax.experimental import pallas as pl
from jax.experimental.pallas import tpu as pltpu
from jax.experimental.pallas import tpu_sc as plsc

N = 10000
D = 128
E = 320000
NC = 2
NS = 16
NW = NC * NS
CHUNK = 80
AR = 10240
RPT = AR // NS

_mesh = plsc.VectorSubcoreMesh(core_axis_name="c", subcore_axis_name="s")


def _fill(ref, nrows, val):
    v = jnp.full((16,), val, jnp.float32)

    def row(i, _):
        def col(j, _):
            ref[i, pl.ds(j * 16, 16)] = v
            return 0
        return lax.fori_loop(0, D // 16, col, 0)
    lax.fori_loop(0, nrows, row, 0)


def _make_agg(layer1):
    nchunk = (E // NS if layer1 else E // NW) // CHUNK

    def body(feat, srci, dsti, out, acc, *bufs):
        c = lax.axis_index("c")
        s = lax.axis_index("s")
        src_v = bufs[0:4]
        dst_v = bufs[4:8]
        rows = bufs[8:12]
        gsem = bufs[12:16]
        ssem = bufs[16:20]
        base = (s * (E // NS)) if layer1 else ((c * NS + s) * (E // NW))

        _fill(rows[0], CHUNK, 0.0)
        for r in range(RPT // CHUNK):
            pltpu.sync_copy(rows[0], acc.at[pl.ds(s * RPT + r * CHUNK, CHUNK)])
        if layer1:
            for b in range(4):
                _fill(rows[b], CHUNK, 1.0)
        plsc.subcore_barrier()

        def start(ci, b):
            off = base + ci * CHUNK
            pltpu.sync_copy(dsti.at[pl.ds(off, CHUNK)], dst_v[b])

            def gath():
                pltpu.sync_copy(srci.at[pl.ds(off, CHUNK)], src_v[b])
                pltpu.async_copy(feat.at[src_v[b]], rows[b], gsem[b])
            if layer1:
                pl.when(c == 0)(gath)
            else:
                gath()

        def wait_scatter(b):
            pltpu.make_async_copy(rows[b], acc.at[dst_v[b]], ssem[b]).wait()

        def finish(ci, b):
            def wait():
                pltpu.make_async_copy(feat.at[src_v[b]], rows[b],
                                      gsem[b]).wait()
            if layer1:
                pl.when(c == 0)(wait)
            else:
                wait()
            pltpu.async_copy(rows[b], acc.at[dst_v[b]], ssem[b], add=True)

        start(0, 0)
        start(1, 1)

        @pl.loop(0, (nchunk // 4) * 4, step=4)
        def _(ci0):
            for b in range(4):
                ci = ci0 + b
                nxt = ci + 2
                nb = (b + 2) % 4

                @pl.when(nxt < nchunk)
                def _():
                    @pl.when(nxt >= 4)
                    def _():
                        wait_scatter(nb)
                    start(nxt, nb)
                finish(ci, b)

        for k in range((nchunk // 4) * 4, nchunk):
            finish(k, k % 4)
        for k in range(max(nchunk - 4, 0), nchunk):
            wait_scatter(k % 4)

        plsc.subcore_barrier()
        pltpu.sync_copy(acc.at[pl.ds(s * RPT, RPT)],
                        out.at[c, pl.ds(s * RPT, RPT)])

    return pl.kernel(
        body,
        out_type=jax.ShapeDtypeStruct((NC, AR, D), jnp.float32),
        mesh=_mesh,
        scratch_types=(
            pltpu.VMEM_SHARED((AR, D), jnp.float32),
            *(pltpu.VMEM((CHUNK,), jnp.int32) for _ in range(8)),
            *(pltpu.VMEM((CHUNK, D), jnp.float32) for _ in range(4)),
            *(pltpu.SemaphoreType.DMA for _ in range(8)),
        ),
    )


_agg1 = _make_agg(True)
_agg2 = _make_agg(False)

_BLK = 1000


def _make_dense(two_partials, relu):
    def body(p_ref, c_ref, x_ref, wl_ref, wr_ref, b_ref, o_ref):
        if two_partials:
            psum = p_ref[0] + p_ref[1]
        else:
            psum = p_ref[0]
        cnt = jnp.maximum(c_ref[0, :, 0:1], 1.0)
        mean = psum / cnt
        acc = lax.dot_general(mean, wl_ref[...], (((1,), (1,)), ((), ())),
                              preferred_element_type=jnp.float32)
        acc = acc + lax.dot_general(x_ref[...], wr_ref[...],
                                    (((1,), (1,)), ((), ())),
                                    preferred_element_type=jnp.float32)
        acc = acc + b_ref[...]
        if relu:
            acc = jnp.maximum(acc, 0.0)
        o_ref[...] = acc

    np = NC if two_partials else 1
    return pl.pallas_call(
        body,
        grid=(N // _BLK,),
        in_specs=[
            pl.BlockSpec((np, _BLK, D), lambda i: (0, i, 0)),
            pl.BlockSpec((1, _BLK, D), lambda i: (1, i, 0)),
            pl.BlockSpec((_BLK, D), lambda i: (i, 0)),
            pl.BlockSpec((D, D), lambda i: (0, 0)),
            pl.BlockSpec((D, D), lambda i: (0, 0)),
            pl.BlockSpec((1, D), lambda i: (0, 0)),
        ],
        out_specs=pl.BlockSpec((_BLK, D), lambda i: (i, 0)),
        out_shape=jax.ShapeDtypeStruct((N, D), jnp.float32),
    )


_dense1 = _make_dense(False, True)
_dense2 = _make_dense(True, False)


@jax.jit
def kernel(x, edge_index, W1l, W1r, b1, W2l, W2r, b2):
    src = edge_index[0].astype(jnp.int32)
    dst = edge_index[1].astype(jnp.int32)
    p1 = _agg1(x, src, dst)
    h = _dense1(p1[0:1], p1, x, W1l, W1r, b1.reshape(1, D))
    p2 = _agg2(h, src, dst)
    return _dense2(p2, p1, h, W2l, W2r, b2.reshape(1, D))


# speedup vs baseline: 11.6102x; 1.3185x over previous
import jax
import jax.numpy as jnp
from jax import lax
from jax.experimental import pallas as pl
from jax.experimental.pallas import tpu as pltpu
from jax.experimental.pallas import tpu_sc as plsc

N = 10000
D = 128
E = 320000
NC = 2
NS = 16
NW = NC * NS
CHUNK = 80
AR = 10240
RPT = AR // NS

_mesh = plsc.VectorSubcoreMesh(core_axis_name="c", subcore_axis_name="s")


def _fill(ref, nrows, val):
    v = jnp.full((16,), val, jnp.float32)

    def row(i, _):
        def col(j, _):
            ref[i, pl.ds(j * 16, 16)] = v
            return 0
        return lax.fori_loop(0, D // 16, col, 0)
    lax.fori_loop(0, nrows, row, 0)


def _make_agg(layer1):
    nchunk = (E // NS if layer1 else E // NW) // CHUNK

    def body(feat, srci, dsti, out, acc, *bufs):
        c = lax.axis_index("c")
        s = lax.axis_index("s")
        src_v = bufs[0:4]
        dst_v = bufs[4:8]
        rows = bufs[8:12]
        gsem = bufs[12:16]
        ssem = bufs[16:20]
        isem = bufs[20:24]
        base = (s * (E // NS)) if layer1 else ((c * NS + s) * (E // NW))

        _fill(rows[0], CHUNK, 0.0)
        for r in range(RPT // CHUNK):
            pltpu.sync_copy(rows[0], acc.at[pl.ds(s * RPT + r * CHUNK, CHUNK)])
        if layer1:
            for b in range(4):
                _fill(rows[b], CHUNK, 1.0)
        plsc.subcore_barrier()

        def load_idx(ci, b):
            off = base + ci * CHUNK
            pltpu.async_copy(srci.at[pl.ds(off, CHUNK)], src_v[b], isem[b])
            pltpu.async_copy(dsti.at[pl.ds(off, CHUNK)], dst_v[b], isem[b])

        def wait_idx(ci, b):
            off = base + ci * CHUNK
            pltpu.make_async_copy(srci.at[pl.ds(off, CHUNK)], src_v[b],
                                  isem[b]).wait()
            pltpu.make_async_copy(dsti.at[pl.ds(off, CHUNK)], dst_v[b],
                                  isem[b]).wait()

        def issue_gather(b):
            def gath():
                pltpu.async_copy(feat.at[src_v[b]], rows[b], gsem[b])
            if layer1:
                pl.when(c == 0)(gath)
            else:
                gath()

        def wait_gather(b):
            def wait():
                pltpu.make_async_copy(feat.at[src_v[b]], rows[b],
                                      gsem[b]).wait()
            if layer1:
                pl.when(c == 0)(wait)
            else:
                wait()

        def issue_scatter(b):
            pltpu.async_copy(rows[b], acc.at[dst_v[b]], ssem[b], add=True)

        def wait_scatter(b):
            pltpu.make_async_copy(rows[b], acc.at[dst_v[b]], ssem[b]).wait()

        load_idx(0, 0)
        load_idx(1, 1)
        wait_idx(0, 0)
        issue_gather(0)

        M = (nchunk // 4) * 4

        @pl.loop(0, M, step=4)
        def _(ci0):
            for b in range(4):
                ci = ci0 + b

                @pl.when(ci + 2 < nchunk)
                def _():
                    @pl.when(ci >= 2)
                    def _():
                        wait_scatter((b + 2) % 4)
                    load_idx(ci + 2, (b + 2) % 4)

                @pl.when(ci + 1 < nchunk)
                def _():
                    wait_idx(ci + 1, (b + 1) % 4)
                    issue_gather((b + 1) % 4)

                wait_gather(b)
                issue_scatter(b)

        for k in range(M, nchunk):
            kb = k % 4
            if k + 2 < nchunk:
                wait_scatter((kb + 2) % 4)
                load_idx(k + 2, (kb + 2) % 4)
            if k + 1 < nchunk:
                wait_idx(k + 1, (kb + 1) % 4)
                issue_gather((kb + 1) % 4)
            wait_gather(kb)
            issue_scatter(kb)
        for k in range(max(nchunk - 4, 0), nchunk):
            wait_scatter(k % 4)

        plsc.subcore_barrier()
        pltpu.sync_copy(acc.at[pl.ds(s * RPT, RPT)],
                        out.at[c, pl.ds(s * RPT, RPT)])

    return pl.kernel(
        body,
        out_type=jax.ShapeDtypeStruct((NC, AR, D), jnp.float32),
        mesh=_mesh,
        scratch_types=(
            pltpu.VMEM_SHARED((AR, D), jnp.float32),
            *(pltpu.VMEM((CHUNK,), jnp.int32) for _ in range(8)),
            *(pltpu.VMEM((CHUNK, D), jnp.float32) for _ in range(4)),
            *(pltpu.SemaphoreType.DMA for _ in range(12)),
        ),
    )


_agg1 = _make_agg(True)
_agg2 = _make_agg(False)

_BLK = 1000


def _make_dense(two_partials, relu):
    def body(p_ref, c_ref, x_ref, wl_ref, wr_ref, b_ref, o_ref):
        if two_partials:
            psum = p_ref[0] + p_ref[1]
        else:
            psum = p_ref[0]
        cnt = jnp.maximum(c_ref[0, :, 0:1], 1.0)
        mean = psum / cnt
        acc = lax.dot_general(mean, wl_ref[...], (((1,), (1,)), ((), ())),
                              preferred_element_type=jnp.float32)
        acc = acc + lax.dot_general(x_ref[...], wr_ref[...],
                                    (((1,), (1,)), ((), ())),
                                    preferred_element_type=jnp.float32)
        acc = acc + b_ref[...]
        if relu:
            acc = jnp.maximum(acc, 0.0)
        o_ref[...] = acc

    np = NC if two_partials else 1
    return pl.pallas_call(
        body,
        grid=(N // _BLK,),
        in_specs=[
            pl.BlockSpec((np, _BLK, D), lambda i: (0, i, 0)),
            pl.BlockSpec((1, _BLK, D), lambda i: (1, i, 0)),
            pl.BlockSpec((_BLK, D), lambda i: (i, 0)),
            pl.BlockSpec((D, D), lambda i: (0, 0)),
            pl.BlockSpec((D, D), lambda i: (0, 0)),
            pl.BlockSpec((1, D), lambda i: (0, 0)),
        ],
        out_specs=pl.BlockSpec((_BLK, D), lambda i: (i, 0)),
        out_shape=jax.ShapeDtypeStruct((N, D), jnp.float32),
    )


_dense1 = _make_dense(False, True)
_dense2 = _make_dense(True, False)


@jax.jit
def kernel(x, edge_index, W1l, W1r, b1, W2l, W2r, b2):
    src = edge_index[0].astype(jnp.int32)
    dst = edge_index[1].astype(jnp.int32)
    p1 = _agg1(x, src, dst)
    h = _dense1(p1[0:1], p1, x, W1l, W1r, b1.reshape(1, D))
    p2 = _agg2(h, src, dst)
    return _dense2(p2, p1, h, W2l, W2r, b2.reshape(1, D))
